# baseline (speedup 1.0000x reference)
import functools

import jax
import jax.numpy as jnp
import numpy as np
from jax import lax
from jax.experimental import pallas as pl
from jax.experimental.pallas import tpu as pltpu
from jax.experimental.pallas import tpu_sc as plsc

N = 10000
E = 320000
D = 86
DP = 128
EXC = 96
NB = 10
MAXR = 2.5
L = 3

NC = 2
NS = 16
NW = NC * NS
EPW = E // NW
CH = 80
NJ = EPW // CH
STRIPE = 1000

BE = 3200
GE = E // BE
BN = 2000
GN = N // BN
BN2 = 1000
GN2 = (N // 2) // BN2

_SQRT_D = np.sqrt(D).astype(np.float32)


def _mesh():
    return plsc.VectorSubcoreMesh(core_axis_name="c", subcore_axis_name="s")


def _sc_gather_pos(px, py, pz, src, dst):
    evec = jax.ShapeDtypeStruct((E,), jnp.float32)
    fbuf = pltpu.VMEM((CH,), jnp.float32)
    ibuf = pltpu.VMEM((CH,), jnp.int32)

    @functools.partial(
        pl.kernel,
        mesh=_mesh(),
        out_type=[evec] * 6,
        scratch_types=[ibuf] * 4 + [fbuf] * 12
        + [pltpu.SemaphoreType.DMA] * 5,
    )
    def gk(tx, ty, tz, sr_, ds_, *refs):
        outs = refs[0:6]
        ib = (refs[6:8], refs[8:10])
        rows = (refs[10:16], refs[16:22])
        si = (refs[22], refs[23])
        sg = refs[24]
        sw = (refs[25], refs[26])
        tabs = (tx, ty, tz, tx, ty, tz)
        cc = lax.axis_index("c")
        ss = lax.axis_index("s")
        base = (ss * NC + cc) * EPW

        def idx_fetch(j, p):
            off = base + j * CH
            pltpu.async_copy(sr_.at[pl.ds(off, CH)], ib[p][0], si[p])
            pltpu.async_copy(ds_.at[pl.ds(off, CH)], ib[p][1], si[p])

        def chunk(j, p, drain):
            off = base + j * CH
            pltpu.make_async_copy(sr_.at[pl.ds(0, CH)], ib[p][0], si[p]).wait()
            pltpu.make_async_copy(ds_.at[pl.ds(0, CH)], ib[p][1], si[p]).wait()
            if drain:
                for q in range(6):
                    pltpu.make_async_copy(
                        rows[p][q], outs[q].at[pl.ds(0, CH)], sw[p]).wait()
            cs = [pltpu.async_copy(tabs[q].at[ib[p][q // 3]], rows[p][q], sg)
                  for q in range(6)]
            for c in cs:
                c.wait()
            for q in range(6):
                pltpu.async_copy(rows[p][q], outs[q].at[pl.ds(off, CH)], sw[p])

        idx_fetch(0, 0)
        idx_fetch(1, 1)
        chunk(0, 0, False)
        idx_fetch(2, 0)
        chunk(1, 1, False)
        idx_fetch(3, 1)
        chunk(2, 0, True)

        def body(t, carry):
            j = 2 * t + 1
            idx_fetch(j + 1, 0)
            chunk(j, 1, True)
            idx_fetch(j + 2, 1)
            chunk(j + 1, 0, True)
            return carry

        lax.fori_loop(1, (NJ - 3) // 2, body, 0)
        idx_fetch(NJ - 1, 0)
        chunk(NJ - 2, 1, True)
        chunk(NJ - 1, 0, True)
        for p in (1, 0):
            for q in range(6):
                pltpu.make_async_copy(
                    rows[p][q], outs[q].at[pl.ds(0, CH)], sw[p]).wait()

    return gk(px, py, pz, src, dst)


def _sc_gather_qkv(qt, kt, vt, dst, src):
    erows = jax.ShapeDtypeStruct((E, DP), jnp.int32)
    rbuf = pltpu.VMEM((CH, DP), jnp.int32)
    ibuf = pltpu.VMEM((CH,), jnp.int32)

    @functools.partial(
        pl.kernel,
        mesh=_mesh(),
        out_type=[erows, erows, erows],
        scratch_types=[ibuf] * 4 + [rbuf] * 6 + [pltpu.SemaphoreType.DMA] * 5,
    )
    def gk(tq, tk, tv, ds_, sr_, *refs):
        outs = refs[0:3]
        ib = (refs[3:5], refs[5:7])
        rows = (refs[7:10], refs[10:13])
        si = (refs[13], refs[14])
        sg = refs[15]
        sw = (refs[16], refs[17])
        tabs = (tq, tk, tv)
        cc = lax.axis_index("c")
        ss = lax.axis_index("s")
        base = (ss * NC + cc) * EPW

        def idx_fetch(j, p):
            off = base + j * CH
            pltpu.async_copy(ds_.at[pl.ds(off, CH)], ib[p][0], si[p])
            pltpu.async_copy(sr_.at[pl.ds(off, CH)], ib[p][1], si[p])

        def chunk(j, p, drain):
            off = base + j * CH
            pltpu.make_async_copy(ds_.at[pl.ds(0, CH)], ib[p][0], si[p]).wait()
            pltpu.make_async_copy(sr_.at[pl.ds(0, CH)], ib[p][1], si[p]).wait()
            if drain:
                for q in range(3):
                    pltpu.make_async_copy(
                        rows[p][q], outs[q].at[pl.ds(0, CH)], sw[p]).wait()
            cs = [pltpu.async_copy(tabs[q].at[ib[p][min(q, 1)]], rows[p][q], sg)
                  for q in range(3)]
            for c in cs:
                c.wait()
            for q in range(3):
                pltpu.async_copy(rows[p][q], outs[q].at[pl.ds(off, CH)], sw[p])

        idx_fetch(0, 0)
        idx_fetch(1, 1)
        chunk(0, 0, False)
        idx_fetch(2, 0)
        chunk(1, 1, False)
        idx_fetch(3, 1)
        chunk(2, 0, True)

        def body(t, carry):
            j = 2 * t + 1
            idx_fetch(j + 1, 0)
            chunk(j, 1, True)
            idx_fetch(j + 2, 1)
            chunk(j + 1, 0, True)
            return carry

        lax.fori_loop(1, (NJ - 3) // 2, body, 0)
        idx_fetch(NJ - 1, 0)
        chunk(NJ - 2, 1, True)
        chunk(NJ - 1, 0, True)
        for p in (1, 0):
            for q in range(3):
                pltpu.make_async_copy(
                    rows[p][q], outs[q].at[pl.ds(0, CH)], sw[p]).wait()

    return gk(qt, kt, vt, dst, src)


def _sc_scatter(rows, dst, zeros_hbm):
    @functools.partial(
        pl.kernel,
        mesh=_mesh(),
        out_type=jax.ShapeDtypeStruct((NC * N, DP), jnp.float32),
        scratch_types=[
            pltpu.VMEM_SHARED((N, DP), jnp.float32),
            pltpu.VMEM((CH,), jnp.int32),
            pltpu.VMEM((CH,), jnp.int32),
            pltpu.VMEM((CH, DP), jnp.float32),
            pltpu.VMEM((CH, DP), jnp.float32),
            pltpu.SemaphoreType.DMA,
            pltpu.SemaphoreType.DMA,
        ],
    )
    def sk(rh, dh, zh, out, acc, ib0, ib1, rb0, rb1, sl0, sl1):
        ib = (ib0, ib1)
        rb = (rb0, rb1)
        sl = (sl0, sl1)
        cc = lax.axis_index("c")
        ss = lax.axis_index("s")
        base = (ss * NC + cc) * EPW
        row0 = ss * STRIPE

        @pl.when(ss < N // STRIPE)
        def _():
            pltpu.sync_copy(zh.at[pl.ds(row0, STRIPE)], acc.at[pl.ds(row0, STRIPE)])

        plsc.subcore_barrier()

        def fetch(j, p):
            off = base + j * CH
            pltpu.async_copy(dh.at[pl.ds(off, CH)], ib[p], sl[p])
            pltpu.async_copy(rh.at[pl.ds(off, CH)], rb[p], sl[p])

        def sadd(p):
            pltpu.make_async_copy(dh.at[pl.ds(0, CH)], ib[p], sl[p]).wait()
            pltpu.make_async_copy(rh.at[pl.ds(0, CH)], rb[p], sl[p]).wait()
            pltpu.sync_copy(rb[p], acc.at[ib[p]], add=True)

        fetch(0, 0)

        def step(t, carry):
            j = 2 * t
            fetch(j + 1, 1)
            sadd(0)
            fetch(j + 2, 0)
            sadd(1)
            return carry

        lax.fori_loop(0, (NJ - 1) // 2, step, 0)
        sadd(0)
        plsc.subcore_barrier()

        @pl.when(ss < N // STRIPE)
        def _():
            pltpu.sync_copy(
                acc.at[pl.ds(row0, STRIPE)],
                out.at[pl.ds(cc * N + row0, STRIPE)],
            )

    return sk(rows, dst, zeros_hbm)


def _embed(pos8, wemb):
    def body(p_ref, w_ref, o_ref):
        o_ref[...] = jnp.tanh(
            jnp.dot(p_ref[...], w_ref[...], preferred_element_type=jnp.float32)
        )

    return pl.pallas_call(
        body,
        grid=(GN,),
        in_specs=[
            pl.BlockSpec((BN, 8), lambda i: (i, 0)),
            pl.BlockSpec((8, DP), lambda i: (0, 0)),
        ],
        out_specs=pl.BlockSpec((BN, DP), lambda i: (i, 0)),
        out_shape=jax.ShapeDtypeStruct((N, DP), jnp.float32),
    )(pos8, wemb)


def _geom(xs, ys, zs, xd, yd, zd):
    wid = np.float32(MAXR / NB)

    def body(xs_r, ys_r, zs_r, xd_r, yd_r, zd_r, rbf_ref, sh_ref):
        rx = xd_r[0] - xs_r[0]
        ry = yd_r[0] - ys_r[0]
        rz = zd_r[0] - zs_r[0]
        d2 = rx * rx + ry * ry + rz * rz
        dist = jnp.sqrt(d2) + 1e-9
        env = jnp.exp(-d2 / (2.0 * MAXR * MAXR))
        rows = []
        for j in range(16):
            if j < NB:
                cj = np.float32(j * MAXR / (NB - 1))
                rows.append(jnp.exp(-(((dist - cj) / wid) ** 2)) * env)
            else:
                rows.append(jnp.zeros_like(dist))
        rbf_ref[...] = jnp.concatenate(rows, axis=0)
        inv = 1.0 / dist
        x = rx * inv
        y = ry * inv
        z = rz * inv
        x2 = x * x
        y2 = y * y
        z2 = z * z
        sh_ref[...] = jnp.concatenate(
            [
                jnp.ones_like(x), x, y, z,
                x * y, y * z, 0.5 * (3.0 * z2 - 1.0), z * x,
                0.5 * (x2 - y2), y * (3.0 * x2 - y2), x * y * z,
                y * (5.0 * z2 - 1.0), z * (5.0 * z2 - 3.0),
                x * (5.0 * z2 - 1.0), z * (x2 - y2), x * (x2 - 3.0 * y2),
            ],
            axis=0,
        )

    espec = pl.BlockSpec((1, 1, BE), lambda i: (i, 0, 0))
    tspec = pl.BlockSpec((16, BE), lambda i: (0, i))
    tshape = jax.ShapeDtypeStruct((16, E), jnp.float32)
    return pl.pallas_call(
        body,
        grid=(GE,),
        in_specs=[espec] * 6,
        out_specs=[tspec, tspec],
        out_shape=[tshape, tshape],
    )(xs, ys, zs, xd, yd, zd)


def _bf16_bits(a):
    bits = lax.bitcast_convert_type(a, jnp.uint32)
    rnd = ((bits >> 16) & 1) + 0x7FFF
    return (bits + rnd) >> 16


def _qkv(x2, wq, wk, wv):
    def pack(lo, hi):
        w = _bf16_bits(lo) | (_bf16_bits(hi) << 16)
        return lax.bitcast_convert_type(w, jnp.int32)

    def body(x_ref, wq_ref, wk_ref, wv_ref, q_ref, k_ref, v_ref):
        xe = x_ref[:, :DP]
        xo = x_ref[:, DP:]
        for w_ref, o_ref in ((wq_ref, q_ref), (wk_ref, k_ref), (wv_ref, v_ref)):
            w = w_ref[...]
            pe = jnp.dot(xe, w, preferred_element_type=jnp.float32)
            po = jnp.dot(xo, w, preferred_element_type=jnp.float32)
            o_ref[...] = pack(pe, po)

    wspec = pl.BlockSpec((DP, DP), lambda i: (0, 0))
    nspec = pl.BlockSpec((BN2, DP), lambda i: (i, 0))
    nshape = jax.ShapeDtypeStruct((N // 2, DP), jnp.int32)
    return pl.pallas_call(
        body,
        grid=(GN2,),
        in_specs=[pl.BlockSpec((BN2, 2 * DP), lambda i: (i, 0)),
                  wspec, wspec, wspec],
        out_specs=[nspec, nspec, nspec],
        out_shape=[nshape, nshape, nshape],
    )(x2, wq, wk, wv)


def _unpack(w_i32, par):
    wu = lax.bitcast_convert_type(w_i32, jnp.uint32)
    lo = lax.bitcast_convert_type(wu << 16, jnp.float32)
    hi = lax.bitcast_convert_type(wu & jnp.uint32(0xFFFF0000), jnp.float32)
    return jnp.where(par > 0, hi, lo)


def _tmod(t_ref, w_ref):
    return lax.dot_general(
        t_ref[...], w_ref[...], (((0,), (0,)), ((), ())),
        preferred_element_type=jnp.float32,
    )


def _pass_a(qdst, ksrc, pard, pars, rbf_t, wr):
    def body(q_ref, k_ref, pd_ref, ps_ref, r_ref, w_ref, lg_ref, gm_ref, mx_ref):
        i = pl.program_id(0)
        rmod = _tmod(r_ref, w_ref)
        q = _unpack(q_ref[...], pd_ref[...])
        k = _unpack(k_ref[...], ps_ref[...])
        prod = q * k * rmod
        lg = jnp.sum(prod, axis=1, keepdims=True) / _SQRT_D
        lg_ref[...] = lg
        bm = jnp.max(lg)

        @pl.when(i == 0)
        def _():
            mx_ref[0, 0] = bm

        @pl.when(i > 0)
        def _():
            mx_ref[0, 0] = jnp.maximum(mx_ref[0, 0], bm)

        gm_ref[0, 0] = mx_ref[0, 0]

    return pl.pallas_call(
        body,
        grid=(GE,),
        in_specs=[
            pl.BlockSpec((BE, DP), lambda i: (i, 0)),
            pl.BlockSpec((BE, DP), lambda i: (i, 0)),
            pl.BlockSpec((BE, 1), lambda i: (i, 0)),
            pl.BlockSpec((BE, 1), lambda i: (i, 0)),
            pl.BlockSpec((16, BE), lambda i: (0, i)),
            pl.BlockSpec((16, DP), lambda i: (0, 0)),
        ],
        out_specs=[
            pl.BlockSpec((BE, 1), lambda i: (i, 0)),
            pl.BlockSpec(memory_space=pltpu.SMEM),
        ],
        out_shape=[
            jax.ShapeDtypeStruct((E, 1), jnp.float32),
            jax.ShapeDtypeStruct((1, 1), jnp.float32),
        ],
        scratch_shapes=[pltpu.SMEM((1, 1), jnp.float32)],
    )(qdst, ksrc, pard, pars, rbf_t, wr)


def _pass_b(vsrc, pars, rbf_t, sh_t, logits, gmax, wr, wsh):
    def body(v_ref, ps_ref, r_ref, s_ref, lg_ref, gm_ref, wr_ref, ws_ref, o_ref):
        rmod = _tmod(r_ref, wr_ref)
        smod = _tmod(s_ref, ws_ref)
        v = _unpack(v_ref[...], ps_ref[...])
        ve = v * rmod + smod
        ex = jnp.exp(lg_ref[...] - gm_ref[0, 0])
        oh = (lax.broadcasted_iota(jnp.int32, (BE, DP), 1) == EXC).astype(
            jnp.float32
        )
        o_ref[...] = ex * (ve + oh)

    return pl.pallas_call(
        body,
        grid=(GE,),
        in_specs=[
            pl.BlockSpec((BE, DP), lambda i: (i, 0)),
            pl.BlockSpec((BE, 1), lambda i: (i, 0)),
            pl.BlockSpec((16, BE), lambda i: (0, i)),
            pl.BlockSpec((16, BE), lambda i: (0, i)),
            pl.BlockSpec((BE, 1), lambda i: (i, 0)),
            pl.BlockSpec(memory_space=pltpu.SMEM),
            pl.BlockSpec((16, DP), lambda i: (0, 0)),
            pl.BlockSpec((16, DP), lambda i: (0, 0)),
        ],
        out_specs=pl.BlockSpec((BE, DP), lambda i: (i, 0)),
        out_shape=jax.ShapeDtypeStruct((E, DP), jnp.float32),
    )(vsrc, pars, rbf_t, sh_t, logits, gmax, wr, wsh)


def _combine(agg2, x, wo):
    def body(a_ref, x_ref, w_ref, o_ref):
        a = a_ref[0] + a_ref[1]
        den = a[:, EXC:EXC + 1] + 1e-9
        node = a / den
        xn = x_ref[...] + jnp.dot(node, w_ref[...], preferred_element_type=jnp.float32)
        mu = jnp.sum(xn, axis=1, keepdims=True) / D
        msk = (lax.broadcasted_iota(jnp.int32, (BN, DP), 1) < D).astype(jnp.float32)
        dv = (xn - mu) * msk
        var = jnp.sum(dv * dv, axis=1, keepdims=True) / D
        sig = jnp.sqrt(var) + 1e-5
        o_ref[...] = dv / sig

    return pl.pallas_call(
        body,
        grid=(GN,),
        in_specs=[
            pl.BlockSpec((2, BN, DP), lambda i: (0, i, 0)),
            pl.BlockSpec((BN, DP), lambda i: (i, 0)),
            pl.BlockSpec((DP, DP), lambda i: (0, 0)),
        ],
        out_specs=pl.BlockSpec((BN, DP), lambda i: (i, 0)),
        out_shape=jax.ShapeDtypeStruct((N, DP), jnp.float32),
    )(agg2, x, wo)


def _final(x, wout):
    def body(x_ref, w_ref, o_ref):
        o_ref[...] = jnp.dot(x_ref[...], w_ref[...], preferred_element_type=jnp.float32)

    return pl.pallas_call(
        body,
        grid=(GN,),
        in_specs=[
            pl.BlockSpec((BN, DP), lambda i: (i, 0)),
            pl.BlockSpec((DP, D), lambda i: (0, 0)),
        ],
        out_specs=pl.BlockSpec((BN, D), lambda i: (i, 0)),
        out_shape=jax.ShapeDtypeStruct((N, D), jnp.float32),
    )(x, wout)


def kernel(pos, edge_index, Wemb, Wq, Wk, Wv, Wr, Wsh, Wo, Wout):
    f32 = jnp.float32
    src = edge_index[0].astype(jnp.int32)
    dst = edge_index[1].astype(jnp.int32)

    pos8 = jnp.pad(pos, ((0, 0), (0, 5)))
    px = jnp.asarray(pos[:, 0], f32)
    py = jnp.asarray(pos[:, 1], f32)
    pz = jnp.asarray(pos[:, 2], f32)
    wemb = jnp.pad(Wemb, ((0, 5), (0, DP - D)))
    wq = jnp.pad(Wq, ((0, 0), (0, DP - D), (0, DP - D)))
    wk = jnp.pad(Wk, ((0, 0), (0, DP - D), (0, DP - D)))
    wv = jnp.pad(Wv, ((0, 0), (0, DP - D), (0, DP - D)))
    wr = jnp.pad(Wr, ((0, 0), (0, 16 - NB), (0, DP - D)))
    wsh = jnp.pad(Wsh, ((0, 0), (0, 0), (0, DP - D)))
    wo = jnp.pad(Wo, ((0, 0), (0, DP - D), (0, DP - D)))
    wout = jnp.pad(Wout, ((0, DP - D), (0, 0)))
    zeros_dp = jnp.zeros((N, DP), f32)

    x = _embed(pos8, wemb)
    xs, ys, zs, xd, yd, zd = _sc_gather_pos(px, py, pz, src, dst)
    shp3 = (GE, 1, BE)
    rbf_t, sh_t = _geom(
        xs.reshape(shp3), ys.reshape(shp3), zs.reshape(shp3),
        xd.reshape(shp3), yd.reshape(shp3), zd.reshape(shp3),
    )

    dst2 = dst >> 1
    src2 = src >> 1
    pard = (dst & 1).reshape(E, 1)
    pars = (src & 1).reshape(E, 1)

    for i in range(L):
        q, k, v = _qkv(x.reshape(N // 2, 2 * DP), wq[i], wk[i], wv[i])
        qdst, ksrc, vsrc = _sc_gather_qkv(q, k, v, dst2, src2)
        logits, gmax = _pass_a(qdst, ksrc, pard, pars, rbf_t, wr[i])
        rows = _pass_b(vsrc, pars, rbf_t, sh_t, logits, gmax, wr[i], wsh[i])
        agg2 = _sc_scatter(rows, dst, zeros_dp)
        x = _combine(agg2.reshape(2, N, DP), x, wo[i])

    return _final(x, wout)


# speedup vs baseline: 1.1633x; 1.1633x over previous
import functools

import jax
import jax.numpy as jnp
import numpy as np
from jax import lax
from jax.experimental import pallas as pl
from jax.experimental.pallas import tpu as pltpu
from jax.experimental.pallas import tpu_sc as plsc

N = 10000
E = 320000
D = 86
DP = 128
EXC = 96
NB = 10
MAXR = 2.5
L = 3

NC = 2
NS = 16
NW = NC * NS
EPW = E // NW
CH = 80
NJ = EPW // CH
STRIPE = 1000

BE = 3200
GE = E // BE
BN = 2000
GN = N // BN
BN2 = 1000
GN2 = (N // 2) // BN2

_SQRT_D = np.sqrt(D).astype(np.float32)


def _mesh():
    return plsc.VectorSubcoreMesh(core_axis_name="c", subcore_axis_name="s")


def _sc_gather_pos(px, py, pz, src, dst):
    evec = jax.ShapeDtypeStruct((E,), jnp.float32)
    fbuf = pltpu.VMEM((CH,), jnp.float32)
    ibuf = pltpu.VMEM((CH,), jnp.int32)

    @functools.partial(
        pl.kernel,
        mesh=_mesh(),
        out_type=[evec] * 6,
        scratch_types=[ibuf] * 4 + [fbuf] * 12
        + [pltpu.SemaphoreType.DMA] * 5,
    )
    def gk(tx, ty, tz, sr_, ds_, *refs):
        outs = refs[0:6]
        ib = (refs[6:8], refs[8:10])
        rows = (refs[10:16], refs[16:22])
        si = (refs[22], refs[23])
        sg = refs[24]
        sw = (refs[25], refs[26])
        tabs = (tx, ty, tz, tx, ty, tz)
        cc = lax.axis_index("c")
        ss = lax.axis_index("s")
        base = (ss * NC + cc) * EPW

        def idx_fetch(j, p):
            off = base + j * CH
            pltpu.async_copy(sr_.at[pl.ds(off, CH)], ib[p][0], si[p])
            pltpu.async_copy(ds_.at[pl.ds(off, CH)], ib[p][1], si[p])

        def chunk(j, p, drain):
            off = base + j * CH
            pltpu.make_async_copy(sr_.at[pl.ds(0, CH)], ib[p][0], si[p]).wait()
            pltpu.make_async_copy(ds_.at[pl.ds(0, CH)], ib[p][1], si[p]).wait()
            if drain:
                for q in range(6):
                    pltpu.make_async_copy(
                        rows[p][q], outs[q].at[pl.ds(0, CH)], sw[p]).wait()
            cs = [pltpu.async_copy(tabs[q].at[ib[p][q // 3]], rows[p][q], sg)
                  for q in range(6)]
            for c in cs:
                c.wait()
            for q in range(6):
                pltpu.async_copy(rows[p][q], outs[q].at[pl.ds(off, CH)], sw[p])

        idx_fetch(0, 0)
        idx_fetch(1, 1)
        chunk(0, 0, False)
        idx_fetch(2, 0)
        chunk(1, 1, False)
        idx_fetch(3, 1)
        chunk(2, 0, True)

        def body(t, carry):
            j = 2 * t + 1
            idx_fetch(j + 1, 0)
            chunk(j, 1, True)
            idx_fetch(j + 2, 1)
            chunk(j + 1, 0, True)
            return carry

        lax.fori_loop(1, (NJ - 3) // 2, body, 0)
        idx_fetch(NJ - 1, 0)
        chunk(NJ - 2, 1, True)
        chunk(NJ - 1, 0, True)
        for p in (1, 0):
            for q in range(6):
                pltpu.make_async_copy(
                    rows[p][q], outs[q].at[pl.ds(0, CH)], sw[p]).wait()

    return gk(px, py, pz, src, dst)


def _sc_gather_rows(pairs):
    npairs = len(pairs)
    erows = jax.ShapeDtypeStruct((E, DP), jnp.float32)
    rbuf = pltpu.VMEM((CH, DP), jnp.float32)
    ibuf = pltpu.VMEM((CH,), jnp.int32)

    @functools.partial(
        pl.kernel,
        mesh=_mesh(),
        out_type=[erows] * npairs,
        scratch_types=[ibuf] * (2 * npairs) + [rbuf] * (2 * npairs)
        + [pltpu.SemaphoreType.DMA] * 5,
    )
    def gk(*refs):
        tabs = refs[0:npairs]
        idxs = refs[npairs:2 * npairs]
        outs = refs[2 * npairs:3 * npairs]
        sc = refs[3 * npairs:]
        ib = (sc[0:npairs], sc[npairs:2 * npairs])
        rows = (sc[2 * npairs:3 * npairs], sc[3 * npairs:4 * npairs])
        si = (sc[4 * npairs], sc[4 * npairs + 1])
        sg = sc[4 * npairs + 2]
        sw = (sc[4 * npairs + 3], sc[4 * npairs + 4])
        cc = lax.axis_index("c")
        ss = lax.axis_index("s")
        base = (ss * NC + cc) * EPW

        def idx_fetch(j, p):
            off = base + j * CH
            for q in range(npairs):
                pltpu.async_copy(idxs[q].at[pl.ds(off, CH)], ib[p][q], si[p])

        def chunk(j, p, drain):
            off = base + j * CH
            for q in range(npairs):
                pltpu.make_async_copy(
                    idxs[q].at[pl.ds(0, CH)], ib[p][q], si[p]).wait()
            if drain:
                for q in range(npairs):
                    pltpu.make_async_copy(
                        rows[p][q], outs[q].at[pl.ds(0, CH)], sw[p]).wait()
            cs = [pltpu.async_copy(tabs[q].at[ib[p][q]], rows[p][q], sg)
                  for q in range(npairs)]
            for c in cs:
                c.wait()
            for q in range(npairs):
                pltpu.async_copy(rows[p][q], outs[q].at[pl.ds(off, CH)], sw[p])

        idx_fetch(0, 0)
        idx_fetch(1, 1)
        chunk(0, 0, False)
        idx_fetch(2, 0)
        chunk(1, 1, False)
        idx_fetch(3, 1)
        chunk(2, 0, True)

        def body(t, carry):
            j = 2 * t + 1
            idx_fetch(j + 1, 0)
            chunk(j, 1, True)
            idx_fetch(j + 2, 1)
            chunk(j + 1, 0, True)
            return carry

        lax.fori_loop(1, (NJ - 3) // 2, body, 0)
        idx_fetch(NJ - 1, 0)
        chunk(NJ - 2, 1, True)
        chunk(NJ - 1, 0, True)
        for p in (1, 0):
            for q in range(npairs):
                pltpu.make_async_copy(
                    rows[p][q], outs[q].at[pl.ds(0, CH)], sw[p]).wait()

    return gk(*[t for t, _ in pairs], *[i for _, i in pairs])


def _sc_gather2x(ta, ia, tb, ib):
    return _sc_gather_rows([(ta, ia), (tb, ib)])


def _sc_gather1(t, i):
    return _sc_gather_rows([(t, i)])[0]


def _sc_scatter(rows, dst, zeros_hbm):
    @functools.partial(
        pl.kernel,
        mesh=_mesh(),
        out_type=jax.ShapeDtypeStruct((NC * N, DP), jnp.float32),
        scratch_types=[
            pltpu.VMEM_SHARED((N, DP), jnp.float32),
            pltpu.VMEM((CH,), jnp.int32),
            pltpu.VMEM((CH,), jnp.int32),
            pltpu.VMEM((CH, DP), jnp.float32),
            pltpu.VMEM((CH, DP), jnp.float32),
            pltpu.SemaphoreType.DMA,
            pltpu.SemaphoreType.DMA,
        ],
    )
    def sk(rh, dh, zh, out, acc, ib0, ib1, rb0, rb1, sl0, sl1):
        ib = (ib0, ib1)
        rb = (rb0, rb1)
        sl = (sl0, sl1)
        cc = lax.axis_index("c")
        ss = lax.axis_index("s")
        base = (ss * NC + cc) * EPW
        row0 = ss * STRIPE

        @pl.when(ss < N // STRIPE)
        def _():
            pltpu.sync_copy(zh.at[pl.ds(row0, STRIPE)], acc.at[pl.ds(row0, STRIPE)])

        plsc.subcore_barrier()

        def fetch(j, p):
            off = base + j * CH
            pltpu.async_copy(dh.at[pl.ds(off, CH)], ib[p], sl[p])
            pltpu.async_copy(rh.at[pl.ds(off, CH)], rb[p], sl[p])

        def sadd(p):
            pltpu.make_async_copy(dh.at[pl.ds(0, CH)], ib[p], sl[p]).wait()
            pltpu.make_async_copy(rh.at[pl.ds(0, CH)], rb[p], sl[p]).wait()
            pltpu.sync_copy(rb[p], acc.at[ib[p]], add=True)

        fetch(0, 0)

        def step(t, carry):
            j = 2 * t
            fetch(j + 1, 1)
            sadd(0)
            fetch(j + 2, 0)
            sadd(1)
            return carry

        lax.fori_loop(0, (NJ - 1) // 2, step, 0)
        sadd(0)
        plsc.subcore_barrier()

        @pl.when(ss < N // STRIPE)
        def _():
            pltpu.sync_copy(
                acc.at[pl.ds(row0, STRIPE)],
                out.at[pl.ds(cc * N + row0, STRIPE)],
            )

    return sk(rows, dst, zeros_hbm)


def _embed(pos8, wemb):
    def body(p_ref, w_ref, o_ref):
        o_ref[...] = jnp.tanh(
            jnp.dot(p_ref[...], w_ref[...], preferred_element_type=jnp.float32)
        )

    return pl.pallas_call(
        body,
        grid=(GN,),
        in_specs=[
            pl.BlockSpec((BN, 8), lambda i: (i, 0)),
            pl.BlockSpec((8, DP), lambda i: (0, 0)),
        ],
        out_specs=pl.BlockSpec((BN, DP), lambda i: (i, 0)),
        out_shape=jax.ShapeDtypeStruct((N, DP), jnp.float32),
    )(pos8, wemb)


def _geom(xs, ys, zs, xd, yd, zd):
    wid = np.float32(MAXR / NB)

    def body(xs_r, ys_r, zs_r, xd_r, yd_r, zd_r, rbf_ref, sh_ref):
        rx = xd_r[0] - xs_r[0]
        ry = yd_r[0] - ys_r[0]
        rz = zd_r[0] - zs_r[0]
        d2 = rx * rx + ry * ry + rz * rz
        dist = jnp.sqrt(d2) + 1e-9
        env = jnp.exp(-d2 / (2.0 * MAXR * MAXR))
        rows = []
        for j in range(16):
            if j < NB:
                cj = np.float32(j * MAXR / (NB - 1))
                rows.append(jnp.exp(-(((dist - cj) / wid) ** 2)) * env)
            else:
                rows.append(jnp.zeros_like(dist))
        rbf_ref[...] = jnp.concatenate(rows, axis=0)
        inv = 1.0 / dist
        x = rx * inv
        y = ry * inv
        z = rz * inv
        x2 = x * x
        y2 = y * y
        z2 = z * z
        sh_ref[...] = jnp.concatenate(
            [
                jnp.ones_like(x), x, y, z,
                x * y, y * z, 0.5 * (3.0 * z2 - 1.0), z * x,
                0.5 * (x2 - y2), y * (3.0 * x2 - y2), x * y * z,
                y * (5.0 * z2 - 1.0), z * (5.0 * z2 - 3.0),
                x * (5.0 * z2 - 1.0), z * (x2 - y2), x * (x2 - 3.0 * y2),
            ],
            axis=0,
        )

    espec = pl.BlockSpec((1, 1, BE), lambda i: (i, 0, 0))
    tspec = pl.BlockSpec((16, BE), lambda i: (0, i))
    tshape = jax.ShapeDtypeStruct((16, E), jnp.float32)
    return pl.pallas_call(
        body,
        grid=(GE,),
        in_specs=[espec] * 6,
        out_specs=[tspec, tspec],
        out_shape=[tshape, tshape],
    )(xs, ys, zs, xd, yd, zd)


def _qkv(x, wq, wk, wv):
    def body(x_ref, wq_ref, wk_ref, wv_ref, q_ref, k_ref, v_ref):
        xv = x_ref[...]
        q_ref[...] = jnp.dot(xv, wq_ref[...], preferred_element_type=jnp.float32)
        k_ref[...] = jnp.dot(xv, wk_ref[...], preferred_element_type=jnp.float32)
        v_ref[...] = jnp.dot(xv, wv_ref[...], preferred_element_type=jnp.float32)

    wspec = pl.BlockSpec((DP, DP), lambda i: (0, 0))
    nspec = pl.BlockSpec((BN, DP), lambda i: (i, 0))
    nshape = jax.ShapeDtypeStruct((N, DP), jnp.float32)
    return pl.pallas_call(
        body,
        grid=(GN,),
        in_specs=[nspec, wspec, wspec, wspec],
        out_specs=[nspec, nspec, nspec],
        out_shape=[nshape, nshape, nshape],
    )(x, wq, wk, wv)


def _tmod(t_ref, w_ref):
    return lax.dot_general(
        t_ref[...], w_ref[...], (((0,), (0,)), ((), ())),
        preferred_element_type=jnp.float32,
    )


def _pass_a(qdst, ksrc, rbf_t, wr):
    def body(q_ref, k_ref, r_ref, w_ref, lg_ref, gm_ref, mx_ref):
        i = pl.program_id(0)
        rmod = _tmod(r_ref, w_ref)
        prod = q_ref[...] * k_ref[...] * rmod
        lg = jnp.sum(prod, axis=1, keepdims=True) / _SQRT_D
        lg_ref[...] = lg
        bm = jnp.max(lg)

        @pl.when(i == 0)
        def _():
            mx_ref[0, 0] = bm

        @pl.when(i > 0)
        def _():
            mx_ref[0, 0] = jnp.maximum(mx_ref[0, 0], bm)

        gm_ref[0, 0] = mx_ref[0, 0]

    return pl.pallas_call(
        body,
        grid=(GE,),
        in_specs=[
            pl.BlockSpec((BE, DP), lambda i: (i, 0)),
            pl.BlockSpec((BE, DP), lambda i: (i, 0)),
            pl.BlockSpec((16, BE), lambda i: (0, i)),
            pl.BlockSpec((16, DP), lambda i: (0, 0)),
        ],
        out_specs=[
            pl.BlockSpec((BE, 1), lambda i: (i, 0)),
            pl.BlockSpec(memory_space=pltpu.SMEM),
        ],
        out_shape=[
            jax.ShapeDtypeStruct((E, 1), jnp.float32),
            jax.ShapeDtypeStruct((1, 1), jnp.float32),
        ],
        scratch_shapes=[pltpu.SMEM((1, 1), jnp.float32)],
    )(qdst, ksrc, rbf_t, wr)


def _pass_b(vsrc, rbf_t, sh_t, logits, gmax, wr, wsh):
    def body(v_ref, r_ref, s_ref, lg_ref, gm_ref, wr_ref, ws_ref, o_ref):
        rmod = _tmod(r_ref, wr_ref)
        smod = _tmod(s_ref, ws_ref)
        ve = v_ref[...] * rmod + smod
        ex = jnp.exp(lg_ref[...] - gm_ref[0, 0])
        oh = (lax.broadcasted_iota(jnp.int32, (BE, DP), 1) == EXC).astype(
            jnp.float32
        )
        o_ref[...] = ex * (ve + oh)

    return pl.pallas_call(
        body,
        grid=(GE,),
        in_specs=[
            pl.BlockSpec((BE, DP), lambda i: (i, 0)),
            pl.BlockSpec((16, BE), lambda i: (0, i)),
            pl.BlockSpec((16, BE), lambda i: (0, i)),
            pl.BlockSpec((BE, 1), lambda i: (i, 0)),
            pl.BlockSpec(memory_space=pltpu.SMEM),
            pl.BlockSpec((16, DP), lambda i: (0, 0)),
            pl.BlockSpec((16, DP), lambda i: (0, 0)),
        ],
        out_specs=pl.BlockSpec((BE, DP), lambda i: (i, 0)),
        out_shape=jax.ShapeDtypeStruct((E, DP), jnp.float32),
    )(vsrc, rbf_t, sh_t, logits, gmax, wr, wsh)


def _combine(agg2, x, wo):
    def body(a_ref, x_ref, w_ref, o_ref):
        a = a_ref[0] + a_ref[1]
        den = a[:, EXC:EXC + 1] + 1e-9
        node = a / den
        xn = x_ref[...] + jnp.dot(node, w_ref[...], preferred_element_type=jnp.float32)
        mu = jnp.sum(xn, axis=1, keepdims=True) / D
        msk = (lax.broadcasted_iota(jnp.int32, (BN, DP), 1) < D).astype(jnp.float32)
        dv = (xn - mu) * msk
        var = jnp.sum(dv * dv, axis=1, keepdims=True) / D
        sig = jnp.sqrt(var) + 1e-5
        o_ref[...] = dv / sig

    return pl.pallas_call(
        body,
        grid=(GN,),
        in_specs=[
            pl.BlockSpec((2, BN, DP), lambda i: (0, i, 0)),
            pl.BlockSpec((BN, DP), lambda i: (i, 0)),
            pl.BlockSpec((DP, DP), lambda i: (0, 0)),
        ],
        out_specs=pl.BlockSpec((BN, DP), lambda i: (i, 0)),
        out_shape=jax.ShapeDtypeStruct((N, DP), jnp.float32),
    )(agg2, x, wo)


def _final(x, wout):
    def body(x_ref, w_ref, o_ref):
        o_ref[...] = jnp.dot(x_ref[...], w_ref[...], preferred_element_type=jnp.float32)

    return pl.pallas_call(
        body,
        grid=(GN,),
        in_specs=[
            pl.BlockSpec((BN, DP), lambda i: (i, 0)),
            pl.BlockSpec((DP, D), lambda i: (0, 0)),
        ],
        out_specs=pl.BlockSpec((BN, D), lambda i: (i, 0)),
        out_shape=jax.ShapeDtypeStruct((N, D), jnp.float32),
    )(x, wout)


def kernel(pos, edge_index, Wemb, Wq, Wk, Wv, Wr, Wsh, Wo, Wout):
    f32 = jnp.float32
    src = edge_index[0].astype(jnp.int32)
    dst = edge_index[1].astype(jnp.int32)

    pos8 = jnp.pad(pos, ((0, 0), (0, 5)))
    px = jnp.asarray(pos[:, 0], f32)
    py = jnp.asarray(pos[:, 1], f32)
    pz = jnp.asarray(pos[:, 2], f32)
    wemb = jnp.pad(Wemb, ((0, 5), (0, DP - D)))
    wq = jnp.pad(Wq, ((0, 0), (0, DP - D), (0, DP - D)))
    wk = jnp.pad(Wk, ((0, 0), (0, DP - D), (0, DP - D)))
    wv = jnp.pad(Wv, ((0, 0), (0, DP - D), (0, DP - D)))
    wr = jnp.pad(Wr, ((0, 0), (0, 16 - NB), (0, DP - D)))
    wsh = jnp.pad(Wsh, ((0, 0), (0, 0), (0, DP - D)))
    wo = jnp.pad(Wo, ((0, 0), (0, DP - D), (0, DP - D)))
    wout = jnp.pad(Wout, ((0, DP - D), (0, 0)))
    zeros_dp = jnp.zeros((N, DP), f32)

    x = _embed(pos8, wemb)
    xs, ys, zs, xd, yd, zd = _sc_gather_pos(px, py, pz, src, dst)
    shp3 = (GE, 1, BE)
    rbf_t, sh_t = _geom(
        xs.reshape(shp3), ys.reshape(shp3), zs.reshape(shp3),
        xd.reshape(shp3), yd.reshape(shp3), zd.reshape(shp3),
    )

    for i in range(L):
        q, k, v = _qkv(x, wq[i], wk[i], wv[i])
        qdst, ksrc = _sc_gather2x(q, dst, k, src)
        vsrc = _sc_gather1(v, src)
        logits, gmax = _pass_a(qdst, ksrc, rbf_t, wr[i])
        rows = _pass_b(vsrc, rbf_t, sh_t, logits, gmax, wr[i], wsh[i])
        agg2 = _sc_scatter(rows, dst, zeros_dp)
        x = _combine(agg2.reshape(2, N, DP), x, wo[i])

    return _final(x, wout)


# speedup vs baseline: 1.2474x; 1.0723x over previous
import functools

import jax
import jax.numpy as jnp
import numpy as np
from jax import lax
from jax.experimental import pallas as pl
from jax.experimental.pallas import tpu as pltpu
from jax.experimental.pallas import tpu_sc as plsc

N = 10000
E = 320000
D = 86
DP = 128
EXC = 96
NB = 10
MAXR = 2.5
L = 3

NC = 2
NS = 16
NW = NC * NS
EPW = E // NW
CH = 80
NJ = EPW // CH
STRIPE = 1000

BE = 6400
GE = E // BE
BN = 2000
GN = N // BN
BN2 = 1000
GN2 = (N // 2) // BN2

_SQRT_D = np.sqrt(D).astype(np.float32)


def _mesh():
    return plsc.VectorSubcoreMesh(core_axis_name="c", subcore_axis_name="s")


def _sc_gather_pos(px, py, pz, src, dst):
    evec = jax.ShapeDtypeStruct((E,), jnp.float32)
    fbuf = pltpu.VMEM((CH,), jnp.float32)
    ibuf = pltpu.VMEM((CH,), jnp.int32)

    @functools.partial(
        pl.kernel,
        mesh=_mesh(),
        out_type=[evec] * 6,
        scratch_types=[ibuf] * 4 + [fbuf] * 12
        + [pltpu.SemaphoreType.DMA] * 5,
    )
    def gk(tx, ty, tz, sr_, ds_, *refs):
        outs = refs[0:6]
        ib = (refs[6:8], refs[8:10])
        rows = (refs[10:16], refs[16:22])
        si = (refs[22], refs[23])
        sg = refs[24]
        sw = (refs[25], refs[26])
        tabs = (tx, ty, tz, tx, ty, tz)
        cc = lax.axis_index("c")
        ss = lax.axis_index("s")
        base = (ss * NC + cc) * EPW

        def idx_fetch(j, p):
            off = base + j * CH
            pltpu.async_copy(sr_.at[pl.ds(off, CH)], ib[p][0], si[p])
            pltpu.async_copy(ds_.at[pl.ds(off, CH)], ib[p][1], si[p])

        def chunk(j, p, drain):
            off = base + j * CH
            pltpu.make_async_copy(sr_.at[pl.ds(0, CH)], ib[p][0], si[p]).wait()
            pltpu.make_async_copy(ds_.at[pl.ds(0, CH)], ib[p][1], si[p]).wait()
            if drain:
                for q in range(6):
                    pltpu.make_async_copy(
                        rows[p][q], outs[q].at[pl.ds(0, CH)], sw[p]).wait()
            cs = [pltpu.async_copy(tabs[q].at[ib[p][q // 3]], rows[p][q], sg)
                  for q in range(6)]
            for c in cs:
                c.wait()
            for q in range(6):
                pltpu.async_copy(rows[p][q], outs[q].at[pl.ds(off, CH)], sw[p])

        idx_fetch(0, 0)
        idx_fetch(1, 1)
        chunk(0, 0, False)
        idx_fetch(2, 0)
        chunk(1, 1, False)
        idx_fetch(3, 1)
        chunk(2, 0, True)

        def body(t, carry):
            j = 2 * t + 1
            idx_fetch(j + 1, 0)
            chunk(j, 1, True)
            idx_fetch(j + 2, 1)
            chunk(j + 1, 0, True)
            return carry

        lax.fori_loop(1, (NJ - 3) // 2, body, 0)
        idx_fetch(NJ - 1, 0)
        chunk(NJ - 2, 1, True)
        chunk(NJ - 1, 0, True)
        for p in (1, 0):
            for q in range(6):
                pltpu.make_async_copy(
                    rows[p][q], outs[q].at[pl.ds(0, CH)], sw[p]).wait()

    return gk(px, py, pz, src, dst)


def _sc_gather_rows(pairs):
    npairs = len(pairs)
    erows = jax.ShapeDtypeStruct((E, DP), jnp.float32)
    rbuf = pltpu.VMEM((CH, DP), jnp.float32)
    ibuf = pltpu.VMEM((CH,), jnp.int32)

    @functools.partial(
        pl.kernel,
        mesh=_mesh(),
        out_type=[erows] * npairs,
        scratch_types=[ibuf] * (2 * npairs) + [rbuf] * (2 * npairs)
        + [pltpu.SemaphoreType.DMA] * 5,
    )
    def gk(*refs):
        tabs = refs[0:npairs]
        idxs = refs[npairs:2 * npairs]
        outs = refs[2 * npairs:3 * npairs]
        sc = refs[3 * npairs:]
        ib = (sc[0:npairs], sc[npairs:2 * npairs])
        rows = (sc[2 * npairs:3 * npairs], sc[3 * npairs:4 * npairs])
        si = (sc[4 * npairs], sc[4 * npairs + 1])
        sg = sc[4 * npairs + 2]
        sw = (sc[4 * npairs + 3], sc[4 * npairs + 4])
        cc = lax.axis_index("c")
        ss = lax.axis_index("s")
        base = (ss * NC + cc) * EPW

        def idx_fetch(j, p):
            off = base + j * CH
            for q in range(npairs):
                pltpu.async_copy(idxs[q].at[pl.ds(off, CH)], ib[p][q], si[p])

        def chunk(j, p, drain):
            off = base + j * CH
            for q in range(npairs):
                pltpu.make_async_copy(
                    idxs[q].at[pl.ds(0, CH)], ib[p][q], si[p]).wait()
            if drain:
                for q in range(npairs):
                    pltpu.make_async_copy(
                        rows[p][q], outs[q].at[pl.ds(0, CH)], sw[p]).wait()
            cs = [pltpu.async_copy(tabs[q].at[ib[p][q]], rows[p][q], sg)
                  for q in range(npairs)]
            for c in cs:
                c.wait()
            for q in range(npairs):
                pltpu.async_copy(rows[p][q], outs[q].at[pl.ds(off, CH)], sw[p])

        idx_fetch(0, 0)
        idx_fetch(1, 1)
        chunk(0, 0, False)
        idx_fetch(2, 0)
        chunk(1, 1, False)
        idx_fetch(3, 1)
        chunk(2, 0, True)

        def body(t, carry):
            j = 2 * t + 1
            idx_fetch(j + 1, 0)
            chunk(j, 1, True)
            idx_fetch(j + 2, 1)
            chunk(j + 1, 0, True)
            return carry

        lax.fori_loop(1, (NJ - 3) // 2, body, 0)
        idx_fetch(NJ - 1, 0)
        chunk(NJ - 2, 1, True)
        chunk(NJ - 1, 0, True)
        for p in (1, 0):
            for q in range(npairs):
                pltpu.make_async_copy(
                    rows[p][q], outs[q].at[pl.ds(0, CH)], sw[p]).wait()

    return gk(*[t for t, _ in pairs], *[i for _, i in pairs])


def _sc_gather2x(ta, ia, tb, ib):
    return _sc_gather_rows([(ta, ia), (tb, ib)])


def _sc_gather1(t, i):
    return _sc_gather_rows([(t, i)])[0]


def _sc_scatter(rows, dst, zeros_hbm):
    @functools.partial(
        pl.kernel,
        mesh=_mesh(),
        out_type=jax.ShapeDtypeStruct((NC * N, DP), jnp.float32),
        scratch_types=[
            pltpu.VMEM_SHARED((N, DP), jnp.float32),
            pltpu.VMEM((CH,), jnp.int32),
            pltpu.VMEM((CH,), jnp.int32),
            pltpu.VMEM((CH, DP), jnp.float32),
            pltpu.VMEM((CH, DP), jnp.float32),
            pltpu.SemaphoreType.DMA,
            pltpu.SemaphoreType.DMA,
        ],
    )
    def sk(rh, dh, zh, out, acc, ib0, ib1, rb0, rb1, sl0, sl1):
        ib = (ib0, ib1)
        rb = (rb0, rb1)
        sl = (sl0, sl1)
        cc = lax.axis_index("c")
        ss = lax.axis_index("s")
        base = (ss * NC + cc) * EPW
        row0 = ss * STRIPE

        @pl.when(ss < N // STRIPE)
        def _():
            pltpu.sync_copy(zh.at[pl.ds(row0, STRIPE)], acc.at[pl.ds(row0, STRIPE)])

        plsc.subcore_barrier()

        def fetch(j, p):
            off = base + j * CH
            pltpu.async_copy(dh.at[pl.ds(off, CH)], ib[p], sl[p])
            pltpu.async_copy(rh.at[pl.ds(off, CH)], rb[p], sl[p])

        def sadd(p):
            pltpu.make_async_copy(dh.at[pl.ds(0, CH)], ib[p], sl[p]).wait()
            pltpu.make_async_copy(rh.at[pl.ds(0, CH)], rb[p], sl[p]).wait()
            pltpu.sync_copy(rb[p], acc.at[ib[p]], add=True)

        fetch(0, 0)

        def step(t, carry):
            j = 2 * t
            fetch(j + 1, 1)
            sadd(0)
            fetch(j + 2, 0)
            sadd(1)
            return carry

        lax.fori_loop(0, (NJ - 1) // 2, step, 0)
        sadd(0)
        plsc.subcore_barrier()

        @pl.when(ss < N // STRIPE)
        def _():
            pltpu.sync_copy(
                acc.at[pl.ds(row0, STRIPE)],
                out.at[pl.ds(cc * N + row0, STRIPE)],
            )

    return sk(rows, dst, zeros_hbm)


def _embed(pos8, wemb):
    def body(p_ref, w_ref, o_ref):
        o_ref[...] = jnp.tanh(
            jnp.dot(p_ref[...], w_ref[...], preferred_element_type=jnp.float32)
        )

    return pl.pallas_call(
        body,
        grid=(GN,),
        in_specs=[
            pl.BlockSpec((BN, 8), lambda i: (i, 0)),
            pl.BlockSpec((8, DP), lambda i: (0, 0)),
        ],
        out_specs=pl.BlockSpec((BN, DP), lambda i: (i, 0)),
        out_shape=jax.ShapeDtypeStruct((N, DP), jnp.float32),
    )(pos8, wemb)


def _geom(xs, ys, zs, xd, yd, zd):
    wid = np.float32(MAXR / NB)

    def body(xs_r, ys_r, zs_r, xd_r, yd_r, zd_r, rbf_ref, sh_ref):
        rx = xd_r[0] - xs_r[0]
        ry = yd_r[0] - ys_r[0]
        rz = zd_r[0] - zs_r[0]
        d2 = rx * rx + ry * ry + rz * rz
        dist = jnp.sqrt(d2) + 1e-9
        env = jnp.exp(-d2 / (2.0 * MAXR * MAXR))
        rows = []
        for j in range(16):
            if j < NB:
                cj = np.float32(j * MAXR / (NB - 1))
                rows.append(jnp.exp(-(((dist - cj) / wid) ** 2)) * env)
            else:
                rows.append(jnp.zeros_like(dist))
        rbf_ref[...] = jnp.concatenate(rows, axis=0)
        inv = 1.0 / dist
        x = rx * inv
        y = ry * inv
        z = rz * inv
        x2 = x * x
        y2 = y * y
        z2 = z * z
        sh_ref[...] = jnp.concatenate(
            [
                jnp.ones_like(x), x, y, z,
                x * y, y * z, 0.5 * (3.0 * z2 - 1.0), z * x,
                0.5 * (x2 - y2), y * (3.0 * x2 - y2), x * y * z,
                y * (5.0 * z2 - 1.0), z * (5.0 * z2 - 3.0),
                x * (5.0 * z2 - 1.0), z * (x2 - y2), x * (x2 - 3.0 * y2),
            ],
            axis=0,
        )

    espec = pl.BlockSpec((1, 1, BE), lambda i: (i, 0, 0))
    tspec = pl.BlockSpec((16, BE), lambda i: (0, i))
    tshape = jax.ShapeDtypeStruct((16, E), jnp.float32)
    return pl.pallas_call(
        body,
        grid=(GE,),
        in_specs=[espec] * 6,
        out_specs=[tspec, tspec],
        out_shape=[tshape, tshape],
    )(xs, ys, zs, xd, yd, zd)


def _qkv(x, wq, wk, wv):
    def body(x_ref, wq_ref, wk_ref, wv_ref, q_ref, k_ref, v_ref):
        xv = x_ref[...]
        q_ref[...] = jnp.dot(xv, wq_ref[...], preferred_element_type=jnp.float32)
        k_ref[...] = jnp.dot(xv, wk_ref[...], preferred_element_type=jnp.float32)
        v_ref[...] = jnp.dot(xv, wv_ref[...], preferred_element_type=jnp.float32)

    wspec = pl.BlockSpec((DP, DP), lambda i: (0, 0))
    nspec = pl.BlockSpec((BN, DP), lambda i: (i, 0))
    nshape = jax.ShapeDtypeStruct((N, DP), jnp.float32)
    return pl.pallas_call(
        body,
        grid=(GN,),
        in_specs=[nspec, wspec, wspec, wspec],
        out_specs=[nspec, nspec, nspec],
        out_shape=[nshape, nshape, nshape],
    )(x, wq, wk, wv)


def _tmod(t_ref, w_ref):
    return lax.dot_general(
        t_ref[...], w_ref[...], (((0,), (0,)), ((), ())),
        preferred_element_type=jnp.float32,
    )


def _pass_a(qdst, ksrc, rbf_t, wr):
    def body(q_ref, k_ref, r_ref, w_ref, lg_ref, gm_ref, mx_ref):
        i = pl.program_id(0)
        rmod = _tmod(r_ref, w_ref)
        prod = q_ref[...] * k_ref[...] * rmod
        lg = jnp.sum(prod, axis=1, keepdims=True) / _SQRT_D
        lg_ref[...] = lg
        bm = jnp.max(lg)

        @pl.when(i == 0)
        def _():
            mx_ref[0, 0] = bm

        @pl.when(i > 0)
        def _():
            mx_ref[0, 0] = jnp.maximum(mx_ref[0, 0], bm)

        gm_ref[0, 0] = mx_ref[0, 0]

    return pl.pallas_call(
        body,
        grid=(GE,),
        in_specs=[
            pl.BlockSpec((BE, DP), lambda i: (i, 0)),
            pl.BlockSpec((BE, DP), lambda i: (i, 0)),
            pl.BlockSpec((16, BE), lambda i: (0, i)),
            pl.BlockSpec((16, DP), lambda i: (0, 0)),
        ],
        out_specs=[
            pl.BlockSpec((BE, 1), lambda i: (i, 0)),
            pl.BlockSpec(memory_space=pltpu.SMEM),
        ],
        out_shape=[
            jax.ShapeDtypeStruct((E, 1), jnp.float32),
            jax.ShapeDtypeStruct((1, 1), jnp.float32),
        ],
        scratch_shapes=[pltpu.SMEM((1, 1), jnp.float32)],
    )(qdst, ksrc, rbf_t, wr)


def _pass_b(vsrc, rbf_t, sh_t, logits, gmax, wr, wsh):
    def body(v_ref, r_ref, s_ref, lg_ref, gm_ref, wr_ref, ws_ref, o_ref):
        rmod = _tmod(r_ref, wr_ref)
        smod = _tmod(s_ref, ws_ref)
        ve = v_ref[...] * rmod + smod
        ex = jnp.exp(lg_ref[...] - gm_ref[0, 0])
        oh = (lax.broadcasted_iota(jnp.int32, (BE, DP), 1) == EXC).astype(
            jnp.float32
        )
        o_ref[...] = ex * (ve + oh)

    return pl.pallas_call(
        body,
        grid=(GE,),
        in_specs=[
            pl.BlockSpec((BE, DP), lambda i: (i, 0)),
            pl.BlockSpec((16, BE), lambda i: (0, i)),
            pl.BlockSpec((16, BE), lambda i: (0, i)),
            pl.BlockSpec((BE, 1), lambda i: (i, 0)),
            pl.BlockSpec(memory_space=pltpu.SMEM),
            pl.BlockSpec((16, DP), lambda i: (0, 0)),
            pl.BlockSpec((16, DP), lambda i: (0, 0)),
        ],
        out_specs=pl.BlockSpec((BE, DP), lambda i: (i, 0)),
        out_shape=jax.ShapeDtypeStruct((E, DP), jnp.float32),
    )(vsrc, rbf_t, sh_t, logits, gmax, wr, wsh)


def _combine(agg2, x, wo):
    def body(a_ref, x_ref, w_ref, o_ref):
        a = a_ref[0] + a_ref[1]
        den = a[:, EXC:EXC + 1] + 1e-9
        node = a / den
        xn = x_ref[...] + jnp.dot(node, w_ref[...], preferred_element_type=jnp.float32)
        mu = jnp.sum(xn, axis=1, keepdims=True) / D
        msk = (lax.broadcasted_iota(jnp.int32, (BN, DP), 1) < D).astype(jnp.float32)
        dv = (xn - mu) * msk
        var = jnp.sum(dv * dv, axis=1, keepdims=True) / D
        sig = jnp.sqrt(var) + 1e-5
        o_ref[...] = dv / sig

    return pl.pallas_call(
        body,
        grid=(GN,),
        in_specs=[
            pl.BlockSpec((2, BN, DP), lambda i: (0, i, 0)),
            pl.BlockSpec((BN, DP), lambda i: (i, 0)),
            pl.BlockSpec((DP, DP), lambda i: (0, 0)),
        ],
        out_specs=pl.BlockSpec((BN, DP), lambda i: (i, 0)),
        out_shape=jax.ShapeDtypeStruct((N, DP), jnp.float32),
    )(agg2, x, wo)


def _final(x, wout):
    def body(x_ref, w_ref, o_ref):
        o_ref[...] = jnp.dot(x_ref[...], w_ref[...], preferred_element_type=jnp.float32)

    return pl.pallas_call(
        body,
        grid=(GN,),
        in_specs=[
            pl.BlockSpec((BN, DP), lambda i: (i, 0)),
            pl.BlockSpec((DP, D), lambda i: (0, 0)),
        ],
        out_specs=pl.BlockSpec((BN, D), lambda i: (i, 0)),
        out_shape=jax.ShapeDtypeStruct((N, D), jnp.float32),
    )(x, wout)


def kernel(pos, edge_index, Wemb, Wq, Wk, Wv, Wr, Wsh, Wo, Wout):
    f32 = jnp.float32
    src = edge_index[0].astype(jnp.int32)
    dst = edge_index[1].astype(jnp.int32)

    pos8 = jnp.pad(pos, ((0, 0), (0, 5)))
    px = jnp.asarray(pos[:, 0], f32)
    py = jnp.asarray(pos[:, 1], f32)
    pz = jnp.asarray(pos[:, 2], f32)
    wemb = jnp.pad(Wemb, ((0, 5), (0, DP - D)))
    wq = jnp.pad(Wq, ((0, 0), (0, DP - D), (0, DP - D)))
    wk = jnp.pad(Wk, ((0, 0), (0, DP - D), (0, DP - D)))
    wv = jnp.pad(Wv, ((0, 0), (0, DP - D), (0, DP - D)))
    wr = jnp.pad(Wr, ((0, 0), (0, 16 - NB), (0, DP - D)))
    wsh = jnp.pad(Wsh, ((0, 0), (0, 0), (0, DP - D)))
    wo = jnp.pad(Wo, ((0, 0), (0, DP - D), (0, DP - D)))
    wout = jnp.pad(Wout, ((0, DP - D), (0, 0)))
    zeros_dp = jnp.zeros((N, DP), f32)

    x = _embed(pos8, wemb)
    xs, ys, zs, xd, yd, zd = _sc_gather_pos(px, py, pz, src, dst)
    shp3 = (GE, 1, BE)
    rbf_t, sh_t = _geom(
        xs.reshape(shp3), ys.reshape(shp3), zs.reshape(shp3),
        xd.reshape(shp3), yd.reshape(shp3), zd.reshape(shp3),
    )

    for i in range(L):
        q, k, v = _qkv(x, wq[i], wk[i], wv[i])
        qdst, ksrc, vsrc = _sc_gather_rows([(q, dst), (k, src), (v, src)])
        logits, gmax = _pass_a(qdst, ksrc, rbf_t, wr[i])
        rows = _pass_b(vsrc, rbf_t, sh_t, logits, gmax, wr[i], wsh[i])
        agg2 = _sc_scatter(rows, dst, zeros_dp)
        x = _combine(agg2.reshape(2, N, DP), x, wo[i])

    return _final(x, wout)


# speedup vs baseline: 1.2508x; 1.0028x over previous
import functools

import jax
import jax.numpy as jnp
import numpy as np
from jax import lax
from jax.experimental import pallas as pl
from jax.experimental.pallas import tpu as pltpu
from jax.experimental.pallas import tpu_sc as plsc

N = 10000
E = 320000
D = 86
DP = 128
EXC = 96
NB = 10
MAXR = 2.5
L = 3

NC = 2
NS = 16
NW = NC * NS
EPW = E // NW
CH = 80
NJ = EPW // CH
STRIPE = 1000

BE = 12800
GE = E // BE
BN = 2000
GN = N // BN
BN2 = 1000
GN2 = (N // 2) // BN2

_SQRT_D = np.sqrt(D).astype(np.float32)


def _mesh():
    return plsc.VectorSubcoreMesh(core_axis_name="c", subcore_axis_name="s")


def _sc_gather_pos(px, py, pz, src, dst):
    evec = jax.ShapeDtypeStruct((E,), jnp.float32)
    fbuf = pltpu.VMEM((CH,), jnp.float32)
    ibuf = pltpu.VMEM((CH,), jnp.int32)

    @functools.partial(
        pl.kernel,
        mesh=_mesh(),
        out_type=[evec] * 6,
        scratch_types=[ibuf] * 4 + [fbuf] * 12
        + [pltpu.SemaphoreType.DMA] * 5,
    )
    def gk(tx, ty, tz, sr_, ds_, *refs):
        outs = refs[0:6]
        ib = (refs[6:8], refs[8:10])
        rows = (refs[10:16], refs[16:22])
        si = (refs[22], refs[23])
        sg = refs[24]
        sw = (refs[25], refs[26])
        tabs = (tx, ty, tz, tx, ty, tz)
        cc = lax.axis_index("c")
        ss = lax.axis_index("s")
        base = (ss * NC + cc) * EPW

        def idx_fetch(j, p):
            off = base + j * CH
            pltpu.async_copy(sr_.at[pl.ds(off, CH)], ib[p][0], si[p])
            pltpu.async_copy(ds_.at[pl.ds(off, CH)], ib[p][1], si[p])

        def chunk(j, p, drain):
            off = base + j * CH
            pltpu.make_async_copy(sr_.at[pl.ds(0, CH)], ib[p][0], si[p]).wait()
            pltpu.make_async_copy(ds_.at[pl.ds(0, CH)], ib[p][1], si[p]).wait()
            if drain:
                for q in range(6):
                    pltpu.make_async_copy(
                        rows[p][q], outs[q].at[pl.ds(0, CH)], sw[p]).wait()
            cs = [pltpu.async_copy(tabs[q].at[ib[p][q // 3]], rows[p][q], sg)
                  for q in range(6)]
            for c in cs:
                c.wait()
            for q in range(6):
                pltpu.async_copy(rows[p][q], outs[q].at[pl.ds(off, CH)], sw[p])

        idx_fetch(0, 0)
        idx_fetch(1, 1)
        chunk(0, 0, False)
        idx_fetch(2, 0)
        chunk(1, 1, False)
        idx_fetch(3, 1)
        chunk(2, 0, True)

        def body(t, carry):
            j = 2 * t + 1
            idx_fetch(j + 1, 0)
            chunk(j, 1, True)
            idx_fetch(j + 2, 1)
            chunk(j + 1, 0, True)
            return carry

        lax.fori_loop(1, (NJ - 3) // 2, body, 0)
        idx_fetch(NJ - 1, 0)
        chunk(NJ - 2, 1, True)
        chunk(NJ - 1, 0, True)
        for p in (1, 0):
            for q in range(6):
                pltpu.make_async_copy(
                    rows[p][q], outs[q].at[pl.ds(0, CH)], sw[p]).wait()

    return gk(px, py, pz, src, dst)


def _sc_gather_rows(pairs):
    npairs = len(pairs)
    erows = jax.ShapeDtypeStruct((E, DP), jnp.float32)
    rbuf = pltpu.VMEM((CH, DP), jnp.float32)
    ibuf = pltpu.VMEM((CH,), jnp.int32)

    @functools.partial(
        pl.kernel,
        mesh=_mesh(),
        out_type=[erows] * npairs,
        scratch_types=[ibuf] * (2 * npairs) + [rbuf] * (2 * npairs)
        + [pltpu.SemaphoreType.DMA] * 5,
    )
    def gk(*refs):
        tabs = refs[0:npairs]
        idxs = refs[npairs:2 * npairs]
        outs = refs[2 * npairs:3 * npairs]
        sc = refs[3 * npairs:]
        ib = (sc[0:npairs], sc[npairs:2 * npairs])
        rows = (sc[2 * npairs:3 * npairs], sc[3 * npairs:4 * npairs])
        si = (sc[4 * npairs], sc[4 * npairs + 1])
        sg = sc[4 * npairs + 2]
        sw = (sc[4 * npairs + 3], sc[4 * npairs + 4])
        cc = lax.axis_index("c")
        ss = lax.axis_index("s")
        base = (ss * NC + cc) * EPW

        def idx_fetch(j, p):
            off = base + j * CH
            for q in range(npairs):
                pltpu.async_copy(idxs[q].at[pl.ds(off, CH)], ib[p][q], si[p])

        def chunk(j, p, drain):
            off = base + j * CH
            for q in range(npairs):
                pltpu.make_async_copy(
                    idxs[q].at[pl.ds(0, CH)], ib[p][q], si[p]).wait()
            if drain:
                for q in range(npairs):
                    pltpu.make_async_copy(
                        rows[p][q], outs[q].at[pl.ds(0, CH)], sw[p]).wait()
            cs = [pltpu.async_copy(tabs[q].at[ib[p][q]], rows[p][q], sg)
                  for q in range(npairs)]
            for c in cs:
                c.wait()
            for q in range(npairs):
                pltpu.async_copy(rows[p][q], outs[q].at[pl.ds(off, CH)], sw[p])

        idx_fetch(0, 0)
        idx_fetch(1, 1)
        chunk(0, 0, False)
        idx_fetch(2, 0)
        chunk(1, 1, False)
        idx_fetch(3, 1)
        chunk(2, 0, True)

        def body(t, carry):
            j = 2 * t + 1
            idx_fetch(j + 1, 0)
            chunk(j, 1, True)
            idx_fetch(j + 2, 1)
            chunk(j + 1, 0, True)
            return carry

        lax.fori_loop(1, (NJ - 3) // 2, body, 0)
        idx_fetch(NJ - 1, 0)
        chunk(NJ - 2, 1, True)
        chunk(NJ - 1, 0, True)
        for p in (1, 0):
            for q in range(npairs):
                pltpu.make_async_copy(
                    rows[p][q], outs[q].at[pl.ds(0, CH)], sw[p]).wait()

    return gk(*[t for t, _ in pairs], *[i for _, i in pairs])


def _sc_gather2x(ta, ia, tb, ib):
    return _sc_gather_rows([(ta, ia), (tb, ib)])


def _sc_gather1(t, i):
    return _sc_gather_rows([(t, i)])[0]


def _sc_scatter(rows, dst, zeros_hbm):
    @functools.partial(
        pl.kernel,
        mesh=_mesh(),
        out_type=jax.ShapeDtypeStruct((NC * N, DP), jnp.float32),
        scratch_types=[
            pltpu.VMEM_SHARED((N, DP), jnp.float32),
            pltpu.VMEM((CH,), jnp.int32),
            pltpu.VMEM((CH,), jnp.int32),
            pltpu.VMEM((CH, DP), jnp.float32),
            pltpu.VMEM((CH, DP), jnp.float32),
            pltpu.SemaphoreType.DMA,
            pltpu.SemaphoreType.DMA,
        ],
    )
    def sk(rh, dh, zh, out, acc, ib0, ib1, rb0, rb1, sl0, sl1):
        ib = (ib0, ib1)
        rb = (rb0, rb1)
        sl = (sl0, sl1)
        cc = lax.axis_index("c")
        ss = lax.axis_index("s")
        base = (ss * NC + cc) * EPW
        row0 = ss * STRIPE

        @pl.when(ss < N // STRIPE)
        def _():
            pltpu.sync_copy(zh.at[pl.ds(row0, STRIPE)], acc.at[pl.ds(row0, STRIPE)])

        plsc.subcore_barrier()

        def fetch(j, p):
            off = base + j * CH
            pltpu.async_copy(dh.at[pl.ds(off, CH)], ib[p], sl[p])
            pltpu.async_copy(rh.at[pl.ds(off, CH)], rb[p], sl[p])

        def sadd(p):
            pltpu.make_async_copy(dh.at[pl.ds(0, CH)], ib[p], sl[p]).wait()
            pltpu.make_async_copy(rh.at[pl.ds(0, CH)], rb[p], sl[p]).wait()
            pltpu.sync_copy(rb[p], acc.at[ib[p]], add=True)

        fetch(0, 0)

        def step(t, carry):
            j = 2 * t
            fetch(j + 1, 1)
            sadd(0)
            fetch(j + 2, 0)
            sadd(1)
            return carry

        lax.fori_loop(0, (NJ - 1) // 2, step, 0)
        sadd(0)
        plsc.subcore_barrier()

        @pl.when(ss < N // STRIPE)
        def _():
            pltpu.sync_copy(
                acc.at[pl.ds(row0, STRIPE)],
                out.at[pl.ds(cc * N + row0, STRIPE)],
            )

    return sk(rows, dst, zeros_hbm)


def _embed(pos8, wemb):
    def body(p_ref, w_ref, o_ref):
        o_ref[...] = jnp.tanh(
            jnp.dot(p_ref[...], w_ref[...], preferred_element_type=jnp.float32)
        )

    return pl.pallas_call(
        body,
        grid=(GN,),
        in_specs=[
            pl.BlockSpec((BN, 8), lambda i: (i, 0)),
            pl.BlockSpec((8, DP), lambda i: (0, 0)),
        ],
        out_specs=pl.BlockSpec((BN, DP), lambda i: (i, 0)),
        out_shape=jax.ShapeDtypeStruct((N, DP), jnp.float32),
    )(pos8, wemb)


def _geom(xs, ys, zs, xd, yd, zd):
    wid = np.float32(MAXR / NB)

    def body(xs_r, ys_r, zs_r, xd_r, yd_r, zd_r, rbf_ref, sh_ref):
        rx = xd_r[0] - xs_r[0]
        ry = yd_r[0] - ys_r[0]
        rz = zd_r[0] - zs_r[0]
        d2 = rx * rx + ry * ry + rz * rz
        dist = jnp.sqrt(d2) + 1e-9
        env = jnp.exp(-d2 / (2.0 * MAXR * MAXR))
        rows = []
        for j in range(16):
            if j < NB:
                cj = np.float32(j * MAXR / (NB - 1))
                rows.append(jnp.exp(-(((dist - cj) / wid) ** 2)) * env)
            else:
                rows.append(jnp.zeros_like(dist))
        rbf_ref[...] = jnp.concatenate(rows, axis=0)
        inv = 1.0 / dist
        x = rx * inv
        y = ry * inv
        z = rz * inv
        x2 = x * x
        y2 = y * y
        z2 = z * z
        sh_ref[...] = jnp.concatenate(
            [
                jnp.ones_like(x), x, y, z,
                x * y, y * z, 0.5 * (3.0 * z2 - 1.0), z * x,
                0.5 * (x2 - y2), y * (3.0 * x2 - y2), x * y * z,
                y * (5.0 * z2 - 1.0), z * (5.0 * z2 - 3.0),
                x * (5.0 * z2 - 1.0), z * (x2 - y2), x * (x2 - 3.0 * y2),
            ],
            axis=0,
        )

    espec = pl.BlockSpec((1, 1, BE), lambda i: (i, 0, 0))
    tspec = pl.BlockSpec((16, BE), lambda i: (0, i))
    tshape = jax.ShapeDtypeStruct((16, E), jnp.float32)
    return pl.pallas_call(
        body,
        grid=(GE,),
        in_specs=[espec] * 6,
        out_specs=[tspec, tspec],
        out_shape=[tshape, tshape],
    )(xs, ys, zs, xd, yd, zd)


def _qkv(x, wq, wk, wv):
    def body(x_ref, wq_ref, wk_ref, wv_ref, q_ref, k_ref, v_ref):
        xv = x_ref[...]
        q_ref[...] = jnp.dot(xv, wq_ref[...], preferred_element_type=jnp.float32)
        k_ref[...] = jnp.dot(xv, wk_ref[...], preferred_element_type=jnp.float32)
        v_ref[...] = jnp.dot(xv, wv_ref[...], preferred_element_type=jnp.float32)

    wspec = pl.BlockSpec((DP, DP), lambda i: (0, 0))
    nspec = pl.BlockSpec((BN, DP), lambda i: (i, 0))
    nshape = jax.ShapeDtypeStruct((N, DP), jnp.float32)
    return pl.pallas_call(
        body,
        grid=(GN,),
        in_specs=[nspec, wspec, wspec, wspec],
        out_specs=[nspec, nspec, nspec],
        out_shape=[nshape, nshape, nshape],
    )(x, wq, wk, wv)


def _tmod(t_ref, w_ref):
    return lax.dot_general(
        t_ref[...], w_ref[...], (((0,), (0,)), ((), ())),
        preferred_element_type=jnp.float32,
    )


def _pass_a(qdst, ksrc, rbf_t, wr):
    def body(q_ref, k_ref, r_ref, w_ref, lg_ref, gm_ref, mx_ref):
        i = pl.program_id(0)
        rmod = _tmod(r_ref, w_ref)
        prod = q_ref[...] * k_ref[...] * rmod
        lg = jnp.sum(prod, axis=1, keepdims=True) / _SQRT_D
        lg_ref[...] = lg
        bm = jnp.max(lg)

        @pl.when(i == 0)
        def _():
            mx_ref[0, 0] = bm

        @pl.when(i > 0)
        def _():
            mx_ref[0, 0] = jnp.maximum(mx_ref[0, 0], bm)

        gm_ref[0, 0] = mx_ref[0, 0]

    return pl.pallas_call(
        body,
        grid=(GE,),
        in_specs=[
            pl.BlockSpec((BE, DP), lambda i: (i, 0)),
            pl.BlockSpec((BE, DP), lambda i: (i, 0)),
            pl.BlockSpec((16, BE), lambda i: (0, i)),
            pl.BlockSpec((16, DP), lambda i: (0, 0)),
        ],
        out_specs=[
            pl.BlockSpec((BE, 1), lambda i: (i, 0)),
            pl.BlockSpec(memory_space=pltpu.SMEM),
        ],
        out_shape=[
            jax.ShapeDtypeStruct((E, 1), jnp.float32),
            jax.ShapeDtypeStruct((1, 1), jnp.float32),
        ],
        scratch_shapes=[pltpu.SMEM((1, 1), jnp.float32)],
    )(qdst, ksrc, rbf_t, wr)


def _pass_b(vsrc, rbf_t, sh_t, logits, gmax, wr, wsh):
    def body(v_ref, r_ref, s_ref, lg_ref, gm_ref, wr_ref, ws_ref, o_ref):
        rmod = _tmod(r_ref, wr_ref)
        smod = _tmod(s_ref, ws_ref)
        ve = v_ref[...] * rmod + smod
        ex = jnp.exp(lg_ref[...] - gm_ref[0, 0])
        oh = (lax.broadcasted_iota(jnp.int32, (BE, DP), 1) == EXC).astype(
            jnp.float32
        )
        o_ref[...] = ex * (ve + oh)

    return pl.pallas_call(
        body,
        grid=(GE,),
        in_specs=[
            pl.BlockSpec((BE, DP), lambda i: (i, 0)),
            pl.BlockSpec((16, BE), lambda i: (0, i)),
            pl.BlockSpec((16, BE), lambda i: (0, i)),
            pl.BlockSpec((BE, 1), lambda i: (i, 0)),
            pl.BlockSpec(memory_space=pltpu.SMEM),
            pl.BlockSpec((16, DP), lambda i: (0, 0)),
            pl.BlockSpec((16, DP), lambda i: (0, 0)),
        ],
        out_specs=pl.BlockSpec((BE, DP), lambda i: (i, 0)),
        out_shape=jax.ShapeDtypeStruct((E, DP), jnp.float32),
    )(vsrc, rbf_t, sh_t, logits, gmax, wr, wsh)


def _combine(agg2, x, wo):
    def body(a_ref, x_ref, w_ref, o_ref):
        a = a_ref[0] + a_ref[1]
        den = a[:, EXC:EXC + 1] + 1e-9
        node = a / den
        xn = x_ref[...] + jnp.dot(node, w_ref[...], preferred_element_type=jnp.float32)
        mu = jnp.sum(xn, axis=1, keepdims=True) / D
        msk = (lax.broadcasted_iota(jnp.int32, (BN, DP), 1) < D).astype(jnp.float32)
        dv = (xn - mu) * msk
        var = jnp.sum(dv * dv, axis=1, keepdims=True) / D
        sig = jnp.sqrt(var) + 1e-5
        o_ref[...] = dv / sig

    return pl.pallas_call(
        body,
        grid=(GN,),
        in_specs=[
            pl.BlockSpec((2, BN, DP), lambda i: (0, i, 0)),
            pl.BlockSpec((BN, DP), lambda i: (i, 0)),
            pl.BlockSpec((DP, DP), lambda i: (0, 0)),
        ],
        out_specs=pl.BlockSpec((BN, DP), lambda i: (i, 0)),
        out_shape=jax.ShapeDtypeStruct((N, DP), jnp.float32),
    )(agg2, x, wo)


def _final(x, wout):
    def body(x_ref, w_ref, o_ref):
        o_ref[...] = jnp.dot(x_ref[...], w_ref[...], preferred_element_type=jnp.float32)

    return pl.pallas_call(
        body,
        grid=(GN,),
        in_specs=[
            pl.BlockSpec((BN, DP), lambda i: (i, 0)),
            pl.BlockSpec((DP, D), lambda i: (0, 0)),
        ],
        out_specs=pl.BlockSpec((BN, D), lambda i: (i, 0)),
        out_shape=jax.ShapeDtypeStruct((N, D), jnp.float32),
    )(x, wout)


def kernel(pos, edge_index, Wemb, Wq, Wk, Wv, Wr, Wsh, Wo, Wout):
    f32 = jnp.float32
    src = edge_index[0].astype(jnp.int32)
    dst = edge_index[1].astype(jnp.int32)

    pos8 = jnp.pad(pos, ((0, 0), (0, 5)))
    px = jnp.asarray(pos[:, 0], f32)
    py = jnp.asarray(pos[:, 1], f32)
    pz = jnp.asarray(pos[:, 2], f32)
    wemb = jnp.pad(Wemb, ((0, 5), (0, DP - D)))
    wq = jnp.pad(Wq, ((0, 0), (0, DP - D), (0, DP - D)))
    wk = jnp.pad(Wk, ((0, 0), (0, DP - D), (0, DP - D)))
    wv = jnp.pad(Wv, ((0, 0), (0, DP - D), (0, DP - D)))
    wr = jnp.pad(Wr, ((0, 0), (0, 16 - NB), (0, DP - D)))
    wsh = jnp.pad(Wsh, ((0, 0), (0, 0), (0, DP - D)))
    wo = jnp.pad(Wo, ((0, 0), (0, DP - D), (0, DP - D)))
    wout = jnp.pad(Wout, ((0, DP - D), (0, 0)))
    zeros_dp = jnp.zeros((N, DP), f32)

    x = _embed(pos8, wemb)
    xs, ys, zs, xd, yd, zd = _sc_gather_pos(px, py, pz, src, dst)
    shp3 = (GE, 1, BE)
    rbf_t, sh_t = _geom(
        xs.reshape(shp3), ys.reshape(shp3), zs.reshape(shp3),
        xd.reshape(shp3), yd.reshape(shp3), zd.reshape(shp3),
    )

    for i in range(L):
        q, k, v = _qkv(x, wq[i], wk[i], wv[i])
        qdst, ksrc, vsrc = _sc_gather_rows([(q, dst), (k, src), (v, src)])
        logits, gmax = _pass_a(qdst, ksrc, rbf_t, wr[i])
        rows = _pass_b(vsrc, rbf_t, sh_t, logits, gmax, wr[i], wsh[i])
        agg2 = _sc_scatter(rows, dst, zeros_dp)
        x = _combine(agg2.reshape(2, N, DP), x, wo[i])

    return _final(x, wout)
